# baseline (speedup 1.0000x reference)
import functools

import jax
import jax.numpy as jnp
from jax import lax
from jax.experimental import pallas as pl
from jax.experimental.pallas import tpu as pltpu
from jax.experimental.pallas import tpu_sc as plsc

B = 16384
F = 26
VOCAB = 100000
DIM = 64

NC = 2
NS = 16
NW = NC * NS
CHUNK = 128
NBLK = B // CHUNK
NPAIR = F // 2
NU = NPAIR * NBLK // NW
NBUF = 2

_mesh = plsc.VectorSubcoreMesh(core_axis_name="c", subcore_axis_name="s")


@functools.partial(
    pl.kernel,
    out_type=jax.ShapeDtypeStruct((B, F * DIM), jnp.float32),
    mesh=_mesh,
    scratch_types=[
        pltpu.VMEM((2 * NU, CHUNK), jnp.int32),
        pltpu.VMEM((NBUF, 2, CHUNK, 2 * DIM), jnp.float32),
        pltpu.VMEM((NBUF, CHUNK, 2 * DIM), jnp.float32),
        pltpu.SemaphoreType.DMA((NBUF,)),
        pltpu.SemaphoreType.DMA((NBUF,)),
    ],
    compiler_params=pltpu.CompilerParams(needs_layout_passes=False),
)
def _gather_rows(tables_hbm, pidx_hbm, out_hbm,
                 pidx_v, bufs, obufs, sem_g, sem_w):
    wid = lax.axis_index("s") * NC + lax.axis_index("c")
    q0 = wid * NU

    pltpu.sync_copy(pidx_hbm.at[pl.ds(2 * q0, 2 * NU)], pidx_v)

    def start_gather(u, b):
        pltpu.async_copy(tables_hbm.at[pidx_v.at[2 * u]], bufs.at[b, 0],
                         sem_g.at[b])
        pltpu.async_copy(tables_hbm.at[pidx_v.at[2 * u + 1]], bufs.at[b, 1],
                         sem_g.at[b])

    def wait_gather(b):
        for h in range(2):
            pltpu.make_async_copy(
                tables_hbm.at[pidx_v.at[0]], bufs.at[b, h], sem_g.at[b]
            ).wait()

    def start_write(u, b):
        q = q0 + u
        fpair = q >> 7
        blk = q & 127
        pltpu.async_copy(
            obufs.at[b],
            out_hbm.at[pl.ds(blk * CHUNK, CHUNK),
                       pl.ds(fpair * 2 * DIM, 2 * DIM)],
            sem_w.at[b],
        )

    def wait_write(b):
        pltpu.make_async_copy(
            obufs.at[b],
            out_hbm.at[pl.ds(0, CHUNK), pl.ds(0, 2 * DIM)],
            sem_w.at[b],
        ).wait()

    def merge(b):
        def row_body(r, carry):
            for k in range(DIM // 16):
                obufs[b, r, pl.ds(k * 16, 16)] = (
                    bufs[b, 0, r, pl.ds(k * 16, 16)]
                )
                obufs[b, r, pl.ds(DIM + k * 16, 16)] = (
                    bufs[b, 1, r, pl.ds(DIM + k * 16, 16)]
                )
            return carry

        lax.fori_loop(0, CHUNK, row_body, 0, unroll=4)

    for b in range(NBUF):
        start_gather(b, b)
    for b in range(NBUF):
        wait_gather(b)
        merge(b)
        start_write(b, b)
        start_gather(NBUF + b, b)

    def outer(k, carry):
        for b in range(NBUF):
            u = k * NBUF + b
            wait_gather(b)
            wait_write(b)
            merge(b)
            start_write(u, b)
            start_gather(u + NBUF, b)
        return carry

    lax.fori_loop(1, NU // NBUF - 1, outer, 0)

    for b in range(NBUF):
        u = NU - NBUF + b
        wait_gather(b)
        wait_write(b)
        merge(b)
        start_write(u, b)
    for b in range(NBUF):
        wait_write(b)


def kernel(x_cat, tables):
    x_cat = x_cat.astype(jnp.int32)
    tables_eo = (
        tables.reshape(NPAIR, 2, VOCAB, DIM)
        .transpose(0, 2, 1, 3)
        .reshape(NPAIR * VOCAB, 2 * DIM)
    )
    off = (jnp.arange(F, dtype=jnp.int32) // 2) * VOCAB
    pidx = x_cat.T + off[:, None]
    pidx = pidx.reshape(NPAIR, 2, NBLK, CHUNK).transpose(0, 2, 1, 3)
    pidx = pidx.reshape(2 * NPAIR * NBLK, CHUNK)
    return _gather_rows(tables_eo, pidx)


# speedup vs baseline: 14.3984x; 14.3984x over previous
import functools

import jax
import jax.numpy as jnp
from jax import lax
from jax.experimental import pallas as pl
from jax.experimental.pallas import tpu as pltpu
from jax.experimental.pallas import tpu_sc as plsc

B = 16384
F = 26
VOCAB = 100000
DIM = 64

NC = 2
NS = 16
NW = NC * NS
CHUNK = 128
NBLK = B // CHUNK
NPAIR = F // 2
NU = NPAIR * NBLK // NW
NBUF = 2

_mesh = plsc.VectorSubcoreMesh(core_axis_name="c", subcore_axis_name="s")


@functools.partial(
    pl.kernel,
    out_type=jax.ShapeDtypeStruct((B, F * DIM), jnp.float32),
    mesh=_mesh,
    scratch_types=[
        pltpu.VMEM((2 * NU, CHUNK), jnp.int32),
        pltpu.VMEM((NBUF, 2, CHUNK), jnp.int32),
        pltpu.VMEM((NBUF, 2, CHUNK, 2 * DIM), jnp.float32),
        pltpu.VMEM((NBUF, CHUNK, 2 * DIM), jnp.float32),
        pltpu.SemaphoreType.DMA((NBUF,)),
        pltpu.SemaphoreType.DMA((NBUF,)),
    ],
    compiler_params=pltpu.CompilerParams(needs_layout_passes=False),
)
def _gather_rows(tables_hbm, pidx_hbm, csel_hbm, out_hbm,
                 pidx_v, csel_u, bufs, obufs, sem_g, sem_w):
    wid = lax.axis_index("s") * NC + lax.axis_index("c")
    q0 = wid * NU

    pltpu.sync_copy(pidx_hbm.at[pl.ds(2 * q0, 2 * NU)], pidx_v)

    def start_gather(u, b):
        pltpu.async_copy(tables_hbm.at[pidx_v.at[2 * u]], bufs.at[b, 0],
                         sem_g.at[b])
        pltpu.async_copy(tables_hbm.at[pidx_v.at[2 * u + 1]], bufs.at[b, 1],
                         sem_g.at[b])
        pltpu.async_copy(csel_hbm.at[pl.ds(2 * (q0 + u), 2)], csel_u.at[b],
                         sem_g.at[b])

    def wait_gather(b):
        for h in range(2):
            pltpu.make_async_copy(
                tables_hbm.at[pidx_v.at[0]], bufs.at[b, h], sem_g.at[b]
            ).wait()
        pltpu.make_async_copy(
            csel_hbm.at[pl.ds(0, 2)], csel_u.at[b], sem_g.at[b]
        ).wait()

    def start_write(u, b):
        q = q0 + u
        fpair = q >> 7
        blk = q & 127
        pltpu.async_copy(
            obufs.at[b],
            out_hbm.at[pl.ds(blk * CHUNK, CHUNK),
                       pl.ds(fpair * 2 * DIM, 2 * DIM)],
            sem_w.at[b],
        )

    def wait_write(b):
        pltpu.make_async_copy(
            obufs.at[b],
            out_hbm.at[pl.ds(0, CHUNK), pl.ds(0, 2 * DIM)],
            sem_w.at[b],
        ).wait()

    zero16 = jnp.zeros((16,), jnp.int32)
    one16 = jnp.ones((16,), jnp.int32)

    def merge(b):
        def row_body(r, carry):
            rv = jnp.full((16,), r, jnp.int32)
            ma = plsc.load_gather(csel_u.at[b], [zero16, rv]) > 0
            mb = plsc.load_gather(csel_u.at[b], [one16, rv]) > 0
            for k in range(DIM // 16):
                lo_a = bufs[b, 0, r, pl.ds(k * 16, 16)]
                hi_a = bufs[b, 0, r, pl.ds(DIM + k * 16, 16)]
                obufs[b, r, pl.ds(k * 16, 16)] = jnp.where(ma, hi_a, lo_a)
                lo_b = bufs[b, 1, r, pl.ds(k * 16, 16)]
                hi_b = bufs[b, 1, r, pl.ds(DIM + k * 16, 16)]
                obufs[b, r, pl.ds(DIM + k * 16, 16)] = jnp.where(mb, hi_b, lo_b)
            return carry

        lax.fori_loop(0, CHUNK, row_body, 0, unroll=2)

    for b in range(NBUF):
        start_gather(b, b)
    for b in range(NBUF):
        wait_gather(b)
        merge(b)
        start_write(b, b)
        start_gather(NBUF + b, b)

    def outer(k, carry):
        for b in range(NBUF):
            u = k * NBUF + b
            wait_gather(b)
            wait_write(b)
            merge(b)
            start_write(u, b)
            start_gather(u + NBUF, b)
        return carry

    lax.fori_loop(1, NU // NBUF - 1, outer, 0)

    for b in range(NBUF):
        u = NU - NBUF + b
        wait_gather(b)
        wait_write(b)
        merge(b)
        start_write(u, b)
    for b in range(NBUF):
        wait_write(b)


def kernel(x_cat, tables):
    x_cat = x_cat.astype(jnp.int32)
    tables_pair = tables.reshape(F * VOCAB // 2, 2 * DIM)
    flat = x_cat.T + (jnp.arange(F, dtype=jnp.int32) * VOCAB)[:, None]
    flat = flat.reshape(NPAIR, 2, NBLK, CHUNK).transpose(0, 2, 1, 3)
    flat = flat.reshape(2 * NPAIR * NBLK, CHUNK)
    pidx = flat >> 1
    csel = flat & 1
    return _gather_rows(tables_pair, pidx, csel)
